# baseline (speedup 1.0000x reference)
import jax
import jax.numpy as jnp
from jax.experimental import pallas as pl

_K = 1000
_N_BISECT = 19


def _mask_kernel(data_ref, wt_ref, b_ref, out_ref):
    x = data_ref[...]
    w = wt_ref[...]
    out = jnp.dot(x, w, preferred_element_type=jnp.float32) + b_ref[...]

    rmax = jnp.max(out, axis=1, keepdims=True)
    rmin = jnp.min(out, axis=1, keepdims=True)
    span = rmax - rmin
    lo0 = rmin - (span * 1e-3 + 1e-30)
    hi0 = rmax

    def body(_, carry):
        lo, hi = carry
        mid = 0.5 * (lo + hi)
        cnt = jnp.sum((out > mid).astype(jnp.float32), axis=1, keepdims=True)
        pred = cnt >= _K
        return jnp.where(pred, mid, lo), jnp.where(pred, hi, mid)

    lo, _ = jax.lax.fori_loop(0, _N_BISECT, body, (lo0, hi0), unroll=4)
    out_ref[...] = jnp.where(out > lo, out, 0.0)


def kernel(data, W, b):
    B, D = data.shape
    V = W.shape[0]
    R = 32 if B % 32 == 0 else (8 if B % 8 == 0 else B)
    wt = W.T
    b2 = b.reshape(1, V)
    return pl.pallas_call(
        _mask_kernel,
        grid=(B // R,),
        in_specs=[
            pl.BlockSpec((R, D), lambda i: (i, 0)),
            pl.BlockSpec((D, V), lambda i: (0, 0)),
            pl.BlockSpec((1, V), lambda i: (0, 0)),
        ],
        out_specs=pl.BlockSpec((R, V), lambda i: (i, 0)),
        out_shape=jax.ShapeDtypeStruct((B, V), jnp.float32),
    )(data, wt, b2)
